# baseline (speedup 1.0000x reference)
import functools

import jax
import jax.numpy as jnp
from jax import lax
from jax.experimental import pallas as pl
from jax.experimental.pallas import tpu as pltpu
from jax.experimental.pallas import tpu_sc as plsc

Q = 1024
K = 100000
D = 128
P = 10
TOPK = 10

KB = 2048
NSTEP = 49
KPAD = NSTEP * KB
SEG = 128
SEGS_PER_STEP = KB // SEG
NSEG = KPAD // SEG
NSEL = 12
NEG = -1e30


def _scores_body(q_ref, k_ref, p_ref, scores_ref, segmax_ref):
    step = pl.program_id(0)
    qs = q_ref[...]
    kb = k_ref[...]
    pj = p_ref[...]
    norm = jnp.sqrt(jnp.sum(pj * pj, axis=1, keepdims=True))
    pjn = pj / (norm + 1e-12)
    cdims = (((1,), (1,)), ((), ()))
    hq = (lax.dot_general(qs, pjn, cdims) > 0).astype(jnp.float32)
    hk = (lax.dot_general(kb, pjn, cdims) > 0).astype(jnp.float32)
    exact = lax.dot_general(qs, kb, cdims)
    hsim = lax.dot_general(hq.astype(jnp.bfloat16), hk.astype(jnp.bfloat16),
                           cdims, preferred_element_type=jnp.float32)
    scores = exact + 0.1 * hsim

    def _emit(sc):
        for j in range(SEGS_PER_STEP):
            scores_ref[j * Q:(j + 1) * Q, :] = sc[:, j * SEG:(j + 1) * SEG]
        segs = [jnp.max(sc[:, i * SEG:(i + 1) * SEG], axis=1, keepdims=True)
                for i in range(SEGS_PER_STEP)]
        segs.append(jnp.full((Q, SEG - SEGS_PER_STEP), NEG, jnp.float32))
        m = jnp.concatenate(segs, axis=1)
        segmax_ref[...] = m.T[:SEGS_PER_STEP, :]

    @pl.when(step < NSTEP - 1)
    def _full():
        _emit(scores)

    @pl.when(step == NSTEP - 1)
    def _masked():
        col = step * KB + lax.broadcasted_iota(jnp.int32, (Q, KB), 1)
        _emit(jnp.where(col < K, scores, NEG))


def _select_body(m_ref, rowid_ref):
    sm = m_ref[...]
    sid = lax.broadcasted_iota(jnp.int32, (NSEG, Q), 0)
    vals = sm
    sels = []
    for _ in range(NSEL):
        mx = jnp.max(vals, axis=0, keepdims=True)
        pick = jnp.min(jnp.where(vals == mx, sid, jnp.int32(NSEG)),
                       axis=0, keepdims=True)
        sels.append(pick)
        vals = jnp.where(sid == pick, NEG, vals)
    sel_t = jnp.concatenate(
        sels + [jnp.zeros((SEG - NSEL, Q), jnp.int32)], axis=0)
    sel_qm = sel_t.T[:, :NSEL]
    qid = lax.broadcasted_iota(jnp.int32, (Q, NSEL), 0)
    rowid_ref[...] = sel_qm * jnp.int32(Q) + qid


def _final_body(g_ref, rid_ref, topi_ref):
    g = g_ref[...]
    rid = rid_ref[...]
    sel = rid // jnp.int32(Q)
    segrep = jnp.concatenate(
        [jnp.broadcast_to(sel[:, j:j + 1], (Q, SEG)) for j in range(NSEL)],
        axis=1)
    t = lax.broadcasted_iota(jnp.int32, (Q, NSEL * SEG), 1) % jnp.int32(SEG)
    gidx = segrep * jnp.int32(SEG) + t
    vals = g
    picks = []
    for _ in range(TOPK):
        mx = jnp.max(vals, axis=1, keepdims=True)
        pick = jnp.min(jnp.where(vals == mx, gidx, jnp.int32(2 ** 30)),
                       axis=1, keepdims=True)
        picks.append(pick)
        vals = jnp.where((vals == mx) & (gidx == pick), NEG, vals)
    topi_ref[...] = jnp.concatenate(
        picks + [jnp.zeros((Q, NSEL - TOPK), jnp.int32)], axis=1)


def _mlp_body(q_ref, c_ref, w1_ref, b1_ref, w2_ref, b2_ref, link_ref):
    qs = q_ref[...]
    w1 = w1_ref[...]
    b1 = b1_ref[...]
    w2 = w2_ref[...]
    b2 = b2_ref[0, 0]
    cols = []
    for j in range(TOPK):
        cj = c_ref[:, j * D:(j + 1) * D]
        pair = jnp.concatenate([qs, cj], axis=1)
        h = jnp.maximum(
            lax.dot_general(pair, w1, (((1,), (0,)), ((), ()))) + b1, 0.0)
        s = jnp.sum(h * w2, axis=1, keepdims=True) + b2
        cols.append(s)
    link = jax.nn.sigmoid(jnp.concatenate(
        cols + [jnp.zeros((Q, NSEL - TOPK), jnp.float32)], axis=1))
    link_ref[...] = link


def _sc_gather(table, idx):
    B = idx.shape[0]
    Dw = table.shape[1]
    NW = 32
    b_per_w = B // NW
    mesh = plsc.VectorSubcoreMesh(core_axis_name="c", subcore_axis_name="s")

    @functools.partial(
        pl.kernel, mesh=mesh,
        out_type=jax.ShapeDtypeStruct((B, Dw), jnp.float32),
        scratch_types=[
            pltpu.VMEM((b_per_w,), jnp.int32),
            pltpu.VMEM((b_per_w, Dw), jnp.float32),
            pltpu.SemaphoreType.DMA,
        ],
    )
    def k(table_hbm, idx_hbm, out_hbm, idx_v, rows_v, sem):
        wid = lax.axis_index("s") * 2 + lax.axis_index("c")
        base = wid * b_per_w
        pltpu.sync_copy(idx_hbm.at[pl.ds(base, b_per_w)], idx_v)
        pltpu.async_copy(table_hbm.at[idx_v], rows_v, sem).wait()
        pltpu.sync_copy(rows_v, out_hbm.at[pl.ds(base, b_per_w)])

    return k(table, idx)


def kernel(queries, keys, projections, W1, b1, W2, b2, top_k):
    proj_pad = jnp.pad(projections, ((0, 16 - P), (0, 0)))

    scores, segmax_t = pl.pallas_call(
        _scores_body,
        grid=(NSTEP,),
        in_specs=[
            pl.BlockSpec((Q, D), lambda k: (0, 0)),
            pl.BlockSpec((KB, D), lambda k: (k, 0)),
            pl.BlockSpec((16, D), lambda k: (0, 0)),
        ],
        out_specs=[
            pl.BlockSpec((SEGS_PER_STEP * Q, SEG), lambda k: (k, 0)),
            pl.BlockSpec((SEGS_PER_STEP, Q), lambda k: (k, 0)),
        ],
        out_shape=[
            jax.ShapeDtypeStruct((NSEG * Q, SEG), jnp.float32),
            jax.ShapeDtypeStruct((NSEG, Q), jnp.float32),
        ],
    )(queries, keys, proj_pad)

    rowid = pl.pallas_call(
        _select_body,
        out_shape=jax.ShapeDtypeStruct((Q, NSEL), jnp.int32),
    )(segmax_t)

    gathered = _sc_gather(scores, rowid.reshape(-1))

    topi16 = pl.pallas_call(
        _final_body,
        out_shape=jax.ShapeDtypeStruct((Q, NSEL), jnp.int32),
    )(gathered.reshape(Q, NSEL * SEG), rowid)
    topi = topi16[:, :TOPK]

    cand = _sc_gather(keys, topi.reshape(-1))

    link16 = pl.pallas_call(
        _mlp_body,
        out_shape=jax.ShapeDtypeStruct((Q, NSEL), jnp.float32),
    )(queries, cand.reshape(Q, TOPK * D), W1, b1.reshape(1, D),
      W2.reshape(1, D), b2.reshape(1, 1))

    return link16[:, :TOPK], topi


# speedup vs baseline: 1.2041x; 1.2041x over previous
import functools

import jax
import jax.numpy as jnp
from jax import lax
from jax.experimental import pallas as pl
from jax.experimental.pallas import tpu as pltpu
from jax.experimental.pallas import tpu_sc as plsc

Q = 1024
K = 100000
D = 128
P = 10
TOPK = 10

KB = 2048
NSTEP = 49
KPAD = NSTEP * KB
SEG = 128
SEGS_PER_STEP = KB // SEG
NSEG = KPAD // SEG
NSEL = 12
NEG = -1e30


def _scores_body(q_ref, k_ref, p_ref, scores_ref, segmax_ref):
    step = pl.program_id(0)
    qs = q_ref[...]
    kb = k_ref[...]
    pj = p_ref[...]
    norm = jnp.sqrt(jnp.sum(pj * pj, axis=1, keepdims=True))
    pjn = pj / (norm + 1e-12)
    cdims = (((1,), (1,)), ((), ()))
    hq = (lax.dot_general(qs, pjn, cdims) > 0).astype(jnp.float32)
    hk = (lax.dot_general(kb, pjn, cdims) > 0).astype(jnp.float32)
    exact = lax.dot_general(qs, kb, cdims)
    hsim = lax.dot_general(hq.astype(jnp.bfloat16), hk.astype(jnp.bfloat16),
                           cdims, preferred_element_type=jnp.float32)
    scores = exact + 0.1 * hsim
    lane = lax.broadcasted_iota(jnp.int32, (1, KB), 1)
    scores = jnp.where(lane < K - step * KB, scores, NEG)
    for j in range(SEGS_PER_STEP):
        scores_ref[j * Q:(j + 1) * Q, :] = scores[:, j * SEG:(j + 1) * SEG]
    segs = [jnp.max(scores[:, i * SEG:(i + 1) * SEG], axis=1, keepdims=True)
            for i in range(SEGS_PER_STEP)]
    segs.append(jnp.full((Q, SEG - SEGS_PER_STEP), NEG, jnp.float32))
    m = jnp.concatenate(segs, axis=1)
    segmax_ref[...] = m.T[:SEGS_PER_STEP, :]


def _select_body(m_ref, rowid_ref):
    sm = m_ref[...]
    sid = lax.broadcasted_iota(jnp.int32, (NSEG, Q), 0)
    vals = sm
    sels = []
    for _ in range(NSEL):
        mx = jnp.max(vals, axis=0, keepdims=True)
        pick = jnp.min(jnp.where(vals == mx, sid, jnp.int32(NSEG)),
                       axis=0, keepdims=True)
        sels.append(pick)
        vals = jnp.where(sid == pick, NEG, vals)
    sel_t = jnp.concatenate(
        sels + [jnp.zeros((SEG - NSEL, Q), jnp.int32)], axis=0)
    sel_qm = sel_t.T[:, :NSEL]
    qid = lax.broadcasted_iota(jnp.int32, (Q, NSEL), 0)
    rowid_ref[...] = sel_qm * jnp.int32(Q) + qid


def _final_body(g_ref, rid_ref, topi_ref):
    g = g_ref[...]
    rid = rid_ref[...]
    sel = rid // jnp.int32(Q)
    segrep = jnp.concatenate(
        [jnp.broadcast_to(sel[:, j:j + 1], (Q, SEG)) for j in range(NSEL)],
        axis=1)
    t = lax.broadcasted_iota(jnp.int32, (Q, NSEL * SEG), 1) % jnp.int32(SEG)
    gidx = segrep * jnp.int32(SEG) + t
    vals = g
    picks = []
    for _ in range(TOPK):
        mx = jnp.max(vals, axis=1, keepdims=True)
        pick = jnp.min(jnp.where(vals == mx, gidx, jnp.int32(2 ** 30)),
                       axis=1, keepdims=True)
        picks.append(pick)
        vals = jnp.where((vals == mx) & (gidx == pick), NEG, vals)
    topi_ref[...] = jnp.concatenate(
        picks + [jnp.zeros((Q, NSEL - TOPK), jnp.int32)], axis=1)


def _mlp_body(q_ref, c_ref, w1_ref, b1_ref, w2_ref, b2_ref, link_ref):
    qs = q_ref[...]
    w1 = w1_ref[...]
    b1 = b1_ref[...]
    w2 = w2_ref[...]
    b2 = b2_ref[0, 0]
    cols = []
    for j in range(TOPK):
        cj = c_ref[:, j * D:(j + 1) * D]
        pair = jnp.concatenate([qs, cj], axis=1)
        h = jnp.maximum(
            lax.dot_general(pair, w1, (((1,), (0,)), ((), ()))) + b1, 0.0)
        s = jnp.sum(h * w2, axis=1, keepdims=True) + b2
        cols.append(s)
    link = jax.nn.sigmoid(jnp.concatenate(
        cols + [jnp.zeros((Q, NSEL - TOPK), jnp.float32)], axis=1))
    link_ref[...] = link


def _sc_gather(table, idx):
    B = idx.shape[0]
    Dw = table.shape[1]
    NW = 32
    b_per_w = B // NW
    mesh = plsc.VectorSubcoreMesh(core_axis_name="c", subcore_axis_name="s")

    @functools.partial(
        pl.kernel, mesh=mesh,
        out_type=jax.ShapeDtypeStruct((B, Dw), jnp.float32),
        scratch_types=[
            pltpu.VMEM((b_per_w,), jnp.int32),
            pltpu.VMEM((b_per_w, Dw), jnp.float32),
            pltpu.SemaphoreType.DMA,
        ],
    )
    def k(table_hbm, idx_hbm, out_hbm, idx_v, rows_v, sem):
        wid = lax.axis_index("s") * 2 + lax.axis_index("c")
        base = wid * b_per_w
        pltpu.sync_copy(idx_hbm.at[pl.ds(base, b_per_w)], idx_v)
        pltpu.async_copy(table_hbm.at[idx_v], rows_v, sem).wait()
        pltpu.sync_copy(rows_v, out_hbm.at[pl.ds(base, b_per_w)])

    return k(table, idx)


def kernel(queries, keys, projections, W1, b1, W2, b2, top_k):
    proj_pad = jnp.pad(projections, ((0, 16 - P), (0, 0)))

    scores, segmax_t = pl.pallas_call(
        _scores_body,
        grid=(NSTEP,),
        in_specs=[
            pl.BlockSpec((Q, D), lambda k: (0, 0)),
            pl.BlockSpec((KB, D), lambda k: (k, 0)),
            pl.BlockSpec((16, D), lambda k: (0, 0)),
        ],
        out_specs=[
            pl.BlockSpec((SEGS_PER_STEP * Q, SEG), lambda k: (k, 0)),
            pl.BlockSpec((SEGS_PER_STEP, Q), lambda k: (k, 0)),
        ],
        out_shape=[
            jax.ShapeDtypeStruct((NSEG * Q, SEG), jnp.float32),
            jax.ShapeDtypeStruct((NSEG, Q), jnp.float32),
        ],
    )(queries, keys, proj_pad)

    rowid = pl.pallas_call(
        _select_body,
        out_shape=jax.ShapeDtypeStruct((Q, NSEL), jnp.int32),
    )(segmax_t)

    gathered = _sc_gather(scores, rowid.reshape(-1))

    topi16 = pl.pallas_call(
        _final_body,
        out_shape=jax.ShapeDtypeStruct((Q, NSEL), jnp.int32),
    )(gathered.reshape(Q, NSEL * SEG), rowid)
    topi = topi16[:, :TOPK]

    cand = _sc_gather(keys, topi.reshape(-1))

    link16 = pl.pallas_call(
        _mlp_body,
        out_shape=jax.ShapeDtypeStruct((Q, NSEL), jnp.float32),
    )(queries, cand.reshape(Q, TOPK * D), W1, b1.reshape(1, D),
      W2.reshape(1, D), b2.reshape(1, 1))

    return link16[:, :TOPK], topi
